# baseline (speedup 1.0000x reference)
import functools

import jax
import jax.numpy as jnp
from jax import lax
from jax.experimental import pallas as pl
from jax.experimental.pallas import tpu as pltpu
from jax.experimental.pallas import tpu_sc as plsc

N_NODES = 10000
N_EDGES = 320000
D_FEAT = 128
CHUNK = 80
GROUPS = CHUNK // 16


def _norms_body(h_ref, out_ref):
    h = h_ref[...]
    out_ref[...] = jnp.sqrt(jnp.sum(h * h, axis=1))


def _node_norms(h):
    return pl.pallas_call(
        _norms_body,
        out_shape=jax.ShapeDtypeStruct((h.shape[0],), jnp.float32),
    )(h)


@functools.cache
def _make_edge_kernel():
    info = plsc.get_sparse_core_info()
    num_cores = info.num_cores
    nw = num_cores * info.num_subcores
    e_per_w = N_EDGES // nw
    n_chunks = e_per_w // CHUNK
    assert n_chunks % 2 == 1

    mesh = plsc.VectorSubcoreMesh(core_axis_name="c", subcore_axis_name="s")

    @functools.partial(
        pl.kernel,
        mesh=mesh,
        compiler_params=pltpu.CompilerParams(needs_layout_passes=False,
                                             use_tc_tiling_on_sc=False),
        out_type=jax.ShapeDtypeStruct((N_EDGES,), jnp.float32),
        scratch_types=[
            pltpu.VMEM((N_NODES,), jnp.float32),
            pltpu.VMEM((e_per_w,), jnp.int32),
            pltpu.VMEM((e_per_w,), jnp.int32),
            pltpu.VMEM((e_per_w,), jnp.float32),
            pltpu.VMEM((CHUNK, D_FEAT // 2), jnp.float32),
            pltpu.VMEM((CHUNK, D_FEAT // 2), jnp.float32),
            pltpu.VMEM((CHUNK, D_FEAT // 2), jnp.float32),
            pltpu.VMEM((CHUNK, D_FEAT // 2), jnp.float32),
            pltpu.VMEM_SHARED((N_NODES, D_FEAT // 2), jnp.float32),
            pltpu.SemaphoreType.DMA,
            pltpu.SemaphoreType.DMA,
        ],
    )
    def edge_kernel(h_hbm, src_hbm, dst_hbm, norms_hbm, out_hbm,
                    norms_v, sids_v, dids_v, out_v,
                    srows0, srows1, drows0, drows1, h_sh,
                    sem0, sem1):
        sub = lax.axis_index("s")
        wid = sub * num_cores + lax.axis_index("c")
        wbase = wid * e_per_w
        slab = 16 * -(-N_NODES // (16 * info.num_subcores))
        last = N_NODES - (info.num_subcores - 1) * slab

        @pl.when(sub < info.num_subcores - 1)
        def _stage_main():
            pltpu.sync_copy(h_hbm.at[pl.ds(sub * slab, slab)],
                            h_sh.at[pl.ds(sub * slab, slab)])

        @pl.when(sub == info.num_subcores - 1)
        def _stage_last():
            base = (info.num_subcores - 1) * slab
            pltpu.sync_copy(h_hbm.at[pl.ds(base, last)],
                            h_sh.at[pl.ds(base, last)])

        pltpu.sync_copy(src_hbm.at[pl.ds(wbase, e_per_w)], sids_v)
        pltpu.sync_copy(dst_hbm.at[pl.ds(wbase, e_per_w)], dids_v)
        pltpu.sync_copy(norms_hbm, norms_v)
        plsc.subcore_barrier()

        def start(c, sbuf, dbuf, sem):
            pltpu.async_copy(h_sh.at[sids_v.at[pl.ds(c * CHUNK, CHUNK)]],
                             sbuf, sem)
            pltpu.async_copy(h_sh.at[dids_v.at[pl.ds(c * CHUNK, CHUNK)]],
                             dbuf, sem)

        def drain(sbuf, dbuf, sem):
            pltpu.make_async_copy(h_hbm.at[pl.ds(0, CHUNK)], sbuf, sem).wait()
            pltpu.make_async_copy(h_hbm.at[pl.ds(0, CHUNK)], dbuf, sem).wait()

        lane = lax.iota(jnp.int32, 16)
        perms = [lane ^ step for step in (8, 4, 2, 1)]
        zero = jnp.zeros((16,), jnp.float32)

        def compute(c, sbuf, dbuf):
            def group_body(g):
                def edge_body(e, num_acc):
                    row = g * 16 + e
                    prods = []
                    for k in range(4):
                        s = plsc.bitcast(sbuf[row, pl.ds(k * 16, 16)],
                                         jnp.bfloat16)
                        t = plsc.bitcast(dbuf[row, pl.ds(k * 16, 16)],
                                         jnp.bfloat16)
                        pa, pb = plsc.unpack(
                            s * t, format=plsc.PackFormat.INTERLEAVED)
                        prods.append(pa)
                        prods.append(pb)
                    acc = ((prods[0] + prods[1]) + (prods[2] + prods[3])) + (
                        (prods[4] + prods[5]) + (prods[6] + prods[7]))
                    for p in perms:
                        acc = acc + acc.at[p].get(mode="promise_in_bounds")
                    m = lane == jnp.broadcast_to(e, (16,))
                    return jnp.where(m, acc, num_acc)

                num_vec = plsc.parallel_loop(0, 16, carry=zero,
                                             unroll=4)(edge_body)
                e0 = c * CHUNK + g * 16
                sid = sids_v[pl.ds(e0, 16)]
                did = dids_v[pl.ds(e0, 16)]
                ns = plsc.load_gather(norms_v, [sid])
                nd = plsc.load_gather(norms_v, [did])
                denom = jnp.maximum(ns * nd, jnp.float32(1e-6))
                out_v[pl.ds(e0, 16)] = num_vec / denom

            plsc.parallel_loop(0, GROUPS)(group_body)

        start(0, srows0, drows0, sem0)

        def pair_body(i, carry):
            c = i * 2
            start(c + 1, srows1, drows1, sem1)
            drain(srows0, drows0, sem0)
            compute(c, srows0, drows0)
            start(c + 2, srows0, drows0, sem0)
            drain(srows1, drows1, sem1)
            compute(c + 1, srows1, drows1)
            return carry

        lax.fori_loop(0, (n_chunks - 1) // 2, pair_body, 0)
        drain(srows0, drows0, sem0)
        compute(n_chunks - 1, srows0, drows0)

        pltpu.sync_copy(out_v, out_hbm.at[pl.ds(wbase, e_per_w)])

    return edge_kernel


def kernel(h, edge_index):
    h = h.astype(jnp.float32)
    ei = edge_index.astype(jnp.int32)
    src = ei[0]
    dst = ei[1]
    norms = _node_norms(h)
    h16 = h.astype(jnp.bfloat16)
    h16w = lax.bitcast_convert_type(
        h16.reshape(N_NODES, D_FEAT // 2, 2), jnp.float32)
    return _make_edge_kernel()(h16w, src, dst, norms)


# speedup vs baseline: 1.0556x; 1.0556x over previous
import functools

import jax
import jax.numpy as jnp
from jax import lax
from jax.experimental import pallas as pl
from jax.experimental.pallas import tpu as pltpu
from jax.experimental.pallas import tpu_sc as plsc

N_NODES = 10000
N_EDGES = 320000
D_FEAT = 128
CHUNK = 80
GROUPS = CHUNK // 16


def _norms_body(h_ref, out_ref):
    h = h_ref[...]
    out_ref[...] = jnp.sqrt(jnp.sum(h * h, axis=1))


def _node_norms(h):
    return pl.pallas_call(
        _norms_body,
        out_shape=jax.ShapeDtypeStruct((h.shape[0],), jnp.float32),
    )(h)


@functools.cache
def _make_edge_kernel():
    info = plsc.get_sparse_core_info()
    num_cores = info.num_cores
    nw = num_cores * info.num_subcores
    e_per_w = N_EDGES // nw
    n_chunks = e_per_w // CHUNK
    assert n_chunks % 2 == 1

    mesh = plsc.VectorSubcoreMesh(core_axis_name="c", subcore_axis_name="s")

    @functools.partial(
        pl.kernel,
        mesh=mesh,
        compiler_params=pltpu.CompilerParams(needs_layout_passes=False,
                                             use_tc_tiling_on_sc=False),
        out_type=jax.ShapeDtypeStruct((N_EDGES,), jnp.float32),
        scratch_types=[
            pltpu.VMEM((N_NODES,), jnp.float32),
            pltpu.VMEM((e_per_w,), jnp.int32),
            pltpu.VMEM((e_per_w,), jnp.int32),
            pltpu.VMEM((e_per_w,), jnp.float32),
            pltpu.VMEM((CHUNK, D_FEAT // 2), jnp.float32),
            pltpu.VMEM((CHUNK, D_FEAT // 2), jnp.float32),
            pltpu.VMEM((CHUNK, D_FEAT // 2), jnp.float32),
            pltpu.VMEM((CHUNK, D_FEAT // 2), jnp.float32),
            pltpu.VMEM_SHARED((N_NODES, D_FEAT // 2), jnp.float32),
            pltpu.SemaphoreType.DMA,
            pltpu.SemaphoreType.DMA,
        ],
    )
    def edge_kernel(h_hbm, src_hbm, dst_hbm, norms_hbm, out_hbm,
                    norms_v, sids_v, dids_v, out_v,
                    srows0, srows1, drows0, drows1, h_sh,
                    sem0, sem1):
        sub = lax.axis_index("s")
        wid = sub * num_cores + lax.axis_index("c")
        wbase = wid * e_per_w
        slab = 16 * -(-N_NODES // (16 * info.num_subcores))
        last = N_NODES - (info.num_subcores - 1) * slab

        @pl.when(sub < info.num_subcores - 1)
        def _stage_main():
            pltpu.sync_copy(h_hbm.at[pl.ds(sub * slab, slab)],
                            h_sh.at[pl.ds(sub * slab, slab)])

        @pl.when(sub == info.num_subcores - 1)
        def _stage_last():
            base = (info.num_subcores - 1) * slab
            pltpu.sync_copy(h_hbm.at[pl.ds(base, last)],
                            h_sh.at[pl.ds(base, last)])

        pltpu.sync_copy(src_hbm.at[pl.ds(wbase, e_per_w)], sids_v)
        pltpu.sync_copy(dst_hbm.at[pl.ds(wbase, e_per_w)], dids_v)
        pltpu.sync_copy(norms_hbm, norms_v)
        plsc.subcore_barrier()

        def start(c, sbuf, dbuf, sem):
            pltpu.async_copy(h_sh.at[sids_v.at[pl.ds(c * CHUNK, CHUNK)]],
                             sbuf, sem)
            pltpu.async_copy(h_sh.at[dids_v.at[pl.ds(c * CHUNK, CHUNK)]],
                             dbuf, sem)

        def drain(sbuf, dbuf, sem):
            pltpu.make_async_copy(h_hbm.at[pl.ds(0, CHUNK)], sbuf, sem).wait()
            pltpu.make_async_copy(h_hbm.at[pl.ds(0, CHUNK)], dbuf, sem).wait()

        lane = lax.iota(jnp.int32, 16)
        perms = [lane ^ step for step in (8, 4, 2, 1)]
        zero = jnp.zeros((16,), jnp.float32)

        def compute(c, sbuf, dbuf):
            def group_body(g):
                def edge_body(e, num_acc):
                    row = g * 16 + e
                    prods = []
                    for k in range(4):
                        s = plsc.bitcast(sbuf[row, pl.ds(k * 16, 16)],
                                         jnp.bfloat16)
                        t = plsc.bitcast(dbuf[row, pl.ds(k * 16, 16)],
                                         jnp.bfloat16)
                        pa, pb = plsc.unpack(
                            s * t, format=plsc.PackFormat.INTERLEAVED)
                        prods.append(pa)
                        prods.append(pb)
                    acc = ((prods[0] + prods[1]) + (prods[2] + prods[3])) + (
                        (prods[4] + prods[5]) + (prods[6] + prods[7]))
                    for p in perms:
                        acc = acc + acc.at[p].get(mode="promise_in_bounds")
                    m = lane == jnp.broadcast_to(e, (16,))
                    return jnp.where(m, acc, num_acc)

                num_vec = plsc.parallel_loop(0, 16, carry=zero,
                                             unroll=2)(edge_body)
                e0 = c * CHUNK + g * 16
                sid = sids_v[pl.ds(e0, 16)]
                did = dids_v[pl.ds(e0, 16)]
                ns = plsc.load_gather(norms_v, [sid])
                nd = plsc.load_gather(norms_v, [did])
                denom = jnp.maximum(ns * nd, jnp.float32(1e-6))
                out_v[pl.ds(e0, 16)] = num_vec / denom

            plsc.parallel_loop(0, GROUPS)(group_body)

        start(0, srows0, drows0, sem0)

        def pair_body(i, carry):
            c = i * 2
            start(c + 1, srows1, drows1, sem1)
            drain(srows0, drows0, sem0)
            compute(c, srows0, drows0)
            start(c + 2, srows0, drows0, sem0)
            drain(srows1, drows1, sem1)
            compute(c + 1, srows1, drows1)
            return carry

        lax.fori_loop(0, (n_chunks - 1) // 2, pair_body, 0)
        drain(srows0, drows0, sem0)
        compute(n_chunks - 1, srows0, drows0)

        pltpu.sync_copy(out_v, out_hbm.at[pl.ds(wbase, e_per_w)])

    return edge_kernel


def kernel(h, edge_index):
    h = h.astype(jnp.float32)
    ei = edge_index.astype(jnp.int32)
    src = ei[0]
    dst = ei[1]
    norms = _node_norms(h)
    h16 = h.astype(jnp.bfloat16)
    h16w = lax.bitcast_convert_type(
        h16.reshape(N_NODES, D_FEAT // 2, 2), jnp.float32)
    return _make_edge_kernel()(h16w, src, dst, norms)


# speedup vs baseline: 1.0806x; 1.0237x over previous
import functools

import jax
import jax.numpy as jnp
from jax import lax
from jax.experimental import pallas as pl
from jax.experimental.pallas import tpu as pltpu
from jax.experimental.pallas import tpu_sc as plsc

N_NODES = 10000
N_EDGES = 320000
D_FEAT = 128
CHUNK = 80
GROUPS = CHUNK // 16


def _norms_body(h_ref, out_ref):
    h = h_ref[...]
    out_ref[...] = jnp.sqrt(jnp.sum(h * h, axis=1))


def _node_norms(h):
    return pl.pallas_call(
        _norms_body,
        out_shape=jax.ShapeDtypeStruct((h.shape[0],), jnp.float32),
    )(h)


@functools.cache
def _make_edge_kernel():
    info = plsc.get_sparse_core_info()
    num_cores = info.num_cores
    nw = num_cores * info.num_subcores
    e_per_w = N_EDGES // nw
    n_chunks = e_per_w // CHUNK
    assert n_chunks % 2 == 1

    mesh = plsc.VectorSubcoreMesh(core_axis_name="c", subcore_axis_name="s")

    @functools.partial(
        pl.kernel,
        mesh=mesh,
        compiler_params=pltpu.CompilerParams(needs_layout_passes=False,
                                             use_tc_tiling_on_sc=False),
        out_type=jax.ShapeDtypeStruct((N_EDGES,), jnp.float32),
        scratch_types=[
            pltpu.VMEM((N_NODES,), jnp.float32),
            pltpu.VMEM((e_per_w,), jnp.int32),
            pltpu.VMEM((e_per_w,), jnp.int32),
            pltpu.VMEM((e_per_w,), jnp.float32),
            pltpu.VMEM((CHUNK, D_FEAT // 2), jnp.float32),
            pltpu.VMEM((CHUNK, D_FEAT // 2), jnp.float32),
            pltpu.VMEM((CHUNK, D_FEAT // 2), jnp.float32),
            pltpu.VMEM((CHUNK, D_FEAT // 2), jnp.float32),
            pltpu.VMEM_SHARED((N_NODES, D_FEAT // 2), jnp.float32),
            pltpu.SemaphoreType.DMA,
            pltpu.SemaphoreType.DMA,
        ],
    )
    def edge_kernel(h_hbm, src_hbm, dst_hbm, norms_hbm, out_hbm,
                    norms_v, sids_v, dids_v, out_v,
                    srows0, srows1, drows0, drows1, h_sh,
                    sem0, sem1):
        sub = lax.axis_index("s")
        wid = sub * num_cores + lax.axis_index("c")
        wbase = wid * e_per_w
        slab = 16 * -(-N_NODES // (16 * info.num_subcores))
        last = N_NODES - (info.num_subcores - 1) * slab

        @pl.when(sub < info.num_subcores - 1)
        def _stage_main():
            pltpu.sync_copy(h_hbm.at[pl.ds(sub * slab, slab)],
                            h_sh.at[pl.ds(sub * slab, slab)])

        @pl.when(sub == info.num_subcores - 1)
        def _stage_last():
            base = (info.num_subcores - 1) * slab
            pltpu.sync_copy(h_hbm.at[pl.ds(base, last)],
                            h_sh.at[pl.ds(base, last)])

        pltpu.sync_copy(src_hbm.at[pl.ds(wbase, e_per_w)], sids_v)
        pltpu.sync_copy(dst_hbm.at[pl.ds(wbase, e_per_w)], dids_v)
        pltpu.sync_copy(norms_hbm, norms_v)
        plsc.subcore_barrier()

        def start(c, sbuf, dbuf, sem):
            pltpu.async_copy(h_sh.at[sids_v.at[pl.ds(c * CHUNK, CHUNK)]],
                             sbuf, sem)
            pltpu.async_copy(h_sh.at[dids_v.at[pl.ds(c * CHUNK, CHUNK)]],
                             dbuf, sem)

        def drain(sbuf, dbuf, sem):
            pltpu.make_async_copy(h_hbm.at[pl.ds(0, CHUNK)], sbuf, sem).wait()
            pltpu.make_async_copy(h_hbm.at[pl.ds(0, CHUNK)], dbuf, sem).wait()

        lane = lax.iota(jnp.int32, 16)
        perm8, perm4, perm2, perm1 = [lane ^ s for s in (8, 4, 2, 1)]
        perm_fix = (lane & 1) << 3
        low_half = lane < 8
        zero = jnp.zeros((16,), jnp.float32)

        def compute(c, sbuf, dbuf):
            def group_body(g):
                def edge_dot(row):
                    prods = []
                    for k in range(4):
                        s = plsc.bitcast(sbuf[row, pl.ds(k * 16, 16)],
                                         jnp.bfloat16)
                        t = plsc.bitcast(dbuf[row, pl.ds(k * 16, 16)],
                                         jnp.bfloat16)
                        pa, pb = plsc.unpack(
                            s * t, format=plsc.PackFormat.INTERLEAVED)
                        prods.append(pa)
                        prods.append(pb)
                    acc = ((prods[0] + prods[1]) + (prods[2] + prods[3])) + (
                        (prods[4] + prods[5]) + (prods[6] + prods[7]))
                    return acc + acc.at[perm8].get(mode="promise_in_bounds")

                def pair_body(i, num_acc):
                    row = g * 16 + i * 2
                    ba = edge_dot(row)
                    bb = edge_dot(row + 1)
                    c2 = jnp.where(low_half, ba, bb)
                    for p in (perm4, perm2, perm1):
                        c2 = c2 + c2.at[p].get(mode="promise_in_bounds")
                    d = c2.at[perm_fix].get(mode="promise_in_bounds")
                    m = (lane >> 1) == jnp.broadcast_to(i, (16,))
                    return jnp.where(m, d, num_acc)

                num_vec = plsc.parallel_loop(0, 8, carry=zero,
                                             unroll=2)(pair_body)
                e0 = c * CHUNK + g * 16
                sid = sids_v[pl.ds(e0, 16)]
                did = dids_v[pl.ds(e0, 16)]
                ns = plsc.load_gather(norms_v, [sid])
                nd = plsc.load_gather(norms_v, [did])
                denom = jnp.maximum(ns * nd, jnp.float32(1e-6))
                out_v[pl.ds(e0, 16)] = num_vec / denom

            plsc.parallel_loop(0, GROUPS)(group_body)

        start(0, srows0, drows0, sem0)

        def pair_body(i, carry):
            c = i * 2
            start(c + 1, srows1, drows1, sem1)
            drain(srows0, drows0, sem0)
            compute(c, srows0, drows0)
            start(c + 2, srows0, drows0, sem0)
            drain(srows1, drows1, sem1)
            compute(c + 1, srows1, drows1)
            return carry

        lax.fori_loop(0, (n_chunks - 1) // 2, pair_body, 0)
        drain(srows0, drows0, sem0)
        compute(n_chunks - 1, srows0, drows0)

        pltpu.sync_copy(out_v, out_hbm.at[pl.ds(wbase, e_per_w)])

    return edge_kernel


def kernel(h, edge_index):
    h = h.astype(jnp.float32)
    ei = edge_index.astype(jnp.int32)
    src = ei[0]
    dst = ei[1]
    norms = _node_norms(h)
    h16 = h.astype(jnp.bfloat16)
    h16w = lax.bitcast_convert_type(
        h16.reshape(N_NODES, D_FEAT // 2, 2), jnp.float32)
    return _make_edge_kernel()(h16w, src, dst, norms)
